# baseline (speedup 1.0000x reference)
import functools

import jax
import jax.numpy as jnp
from jax import lax
from jax.experimental import pallas as pl
from jax.experimental.pallas import tpu as pltpu
from jax.experimental.pallas import tpu_sc as plsc

F32 = jnp.float32
I32 = jnp.int32
NSC = 2
NSUB = 16
NW = NSC * NSUB
CH = 1024
ZCH = 448


def _mesh():
  return plsc.VectorSubcoreMesh(core_axis_name="c", subcore_axis_name="s")


def _ids():
  c = lax.axis_index("c")
  s = lax.axis_index("s")
  return c, s, c * NSUB + s


@functools.cache
def _make_sc_deg(NP, EP):
  TPE = EP // NW
  NSTR = TPE // CH
  STRIPE = NP // NSUB

  def body(dst_hbm, out_hbm, idx_v, ones_v, zbuf_v, acc_sh):
    c, s, w = _ids()

    @pl.loop(0, CH // 16)
    def _(i):
      ones_v[pl.ds(i * 16, 16)] = jnp.ones((16,), F32)

    @pl.loop(0, ZCH // 16)
    def _(i):
      zbuf_v[pl.ds(i * 16, 16)] = jnp.zeros((16,), F32)

    @pl.loop(0, STRIPE // ZCH)
    def _(i):
      pltpu.sync_copy(zbuf_v, acc_sh.at[pl.ds(s * STRIPE + i * ZCH, ZCH)])

    plsc.subcore_barrier()

    @pl.loop(0, NSTR)
    def _(j):
      pltpu.sync_copy(dst_hbm.at[pl.ds(w * TPE + j * CH, CH)], idx_v)
      pltpu.sync_copy(ones_v, acc_sh.at[idx_v], add=True)

    plsc.subcore_barrier()
    pltpu.sync_copy(
        acc_sh.at[pl.ds(s * STRIPE, STRIPE)],
        out_hbm.at[c, pl.ds(s * STRIPE, STRIPE)],
    )

  return pl.kernel(
      body,
      out_type=jax.ShapeDtypeStruct((NSC, NP), F32),
      mesh=_mesh(),
      compiler_params=pltpu.CompilerParams(use_tc_tiling_on_sc=False),
      scratch_types=[
          pltpu.VMEM((CH,), I32),
          pltpu.VMEM((CH,), F32),
          pltpu.VMEM((ZCH,), F32),
          pltpu.VMEM_SHARED((NP,), F32),
      ],
  )


@functools.cache
def _make_sc_edge1(NP, EP):
  TPE = EP // NW
  NSTR = TPE // CH
  STRIPE = NP // NSUB

  def body(xp_hbm, src_hbm, dst_hbm, out_hbm, sidx_v, didx_v, val_v, zbuf_v,
           xp_sh, acc_sh):
    c, s, w = _ids()

    @pl.loop(0, ZCH // 16)
    def _(i):
      zbuf_v[pl.ds(i * 16, 16)] = jnp.zeros((16,), F32)

    pltpu.sync_copy(
        xp_hbm.at[pl.ds(s * STRIPE, STRIPE)],
        xp_sh.at[pl.ds(s * STRIPE, STRIPE)],
    )

    @pl.loop(0, STRIPE // ZCH)
    def _(i):
      pltpu.sync_copy(zbuf_v, acc_sh.at[pl.ds(s * STRIPE + i * ZCH, ZCH)])

    plsc.subcore_barrier()

    @pl.loop(0, NSTR)
    def _(j):
      base = w * TPE + j * CH
      pltpu.sync_copy(src_hbm.at[pl.ds(base, CH)], sidx_v)
      pltpu.sync_copy(xp_sh.at[sidx_v], val_v)
      pltpu.sync_copy(dst_hbm.at[pl.ds(base, CH)], didx_v)
      pltpu.sync_copy(val_v, acc_sh.at[didx_v], add=True)

    plsc.subcore_barrier()
    pltpu.sync_copy(
        acc_sh.at[pl.ds(s * STRIPE, STRIPE)],
        out_hbm.at[c, pl.ds(s * STRIPE, STRIPE)],
    )

  return pl.kernel(
      body,
      out_type=jax.ShapeDtypeStruct((NSC, NP), F32),
      mesh=_mesh(),
      compiler_params=pltpu.CompilerParams(use_tc_tiling_on_sc=False),
      scratch_types=[
          pltpu.VMEM((CH,), I32),
          pltpu.VMEM((CH,), I32),
          pltpu.VMEM((CH,), F32),
          pltpu.VMEM((ZCH,), F32),
          pltpu.VMEM_SHARED((NP,), F32),
          pltpu.VMEM_SHARED((NP,), F32),
      ],
  )


@functools.cache
def _make_sc_edge(NP, EP, HH):
  CHE = 1024
  TPE = EP // NW
  NSTR = TPE // CHE
  STRIPE = NP // NSUB
  ZR = 112

  def body(hp0_hbm, hp1_hbm, src_hbm, dst_hbm, out_hbm, sidx_v, didx_v,
           rows_v, zbuf_v, acc_sh, sem):
    c, s, w = _ids()

    @pl.loop(0, ZR)
    def _(r):
      zbuf_v[r] = jnp.zeros((HH,), F32)

    for f in range(2):
      hp_hbm = hp0_hbm if f == 0 else hp1_hbm

      @pl.loop(0, STRIPE // ZR)
      def _(i):
        pltpu.sync_copy(zbuf_v, acc_sh.at[pl.ds(s * STRIPE + i * ZR, ZR)])

      plsc.subcore_barrier()

      @pl.loop(0, NSTR)
      def _(j):
        base = w * TPE + j * CHE
        pltpu.sync_copy(src_hbm.at[pl.ds(base, CHE)], sidx_v)
        gather = pltpu.async_copy(hp_hbm.at[sidx_v], rows_v, sem)
        pltpu.sync_copy(dst_hbm.at[pl.ds(base, CHE)], didx_v)
        gather.wait()
        pltpu.sync_copy(rows_v, acc_sh.at[didx_v], add=True)

      plsc.subcore_barrier()
      pltpu.sync_copy(
          acc_sh.at[pl.ds(s * STRIPE, STRIPE)],
          out_hbm.at[c, f, pl.ds(s * STRIPE, STRIPE)],
      )
      plsc.subcore_barrier()

  return pl.kernel(
      body,
      out_type=jax.ShapeDtypeStruct((NSC, 2, NP, HH), F32),
      mesh=_mesh(),
      compiler_params=pltpu.CompilerParams(use_tc_tiling_on_sc=False),
      scratch_types=[
          pltpu.VMEM((CHE,), I32),
          pltpu.VMEM((CHE,), I32),
          pltpu.VMEM((CHE, HH), F32),
          pltpu.VMEM((ZR, HH), F32),
          pltpu.VMEM_SHARED((NP, HH), F32),
          pltpu.SemaphoreType.DMA,
      ],
  )


BN = 2048


def _node_spec(shape_tail):
  return pl.BlockSpec((BN,) + shape_tail, lambda i: (i,) + (0,) * len(shape_tail))


def _pair_spec(shape_tail):
  return pl.BlockSpec((NSC, BN) + shape_tail,
                      lambda i: (0, i) + (0,) * len(shape_tail))


def _full_spec(shape):
  return pl.BlockSpec(shape, lambda i: (0,) * len(shape))


def _acc_spec():
  return pl.BlockSpec((NSC, 2, BN, 16), lambda i: (0, 0, i, 0))


def _acc_sum(a_ref):
  return jnp.concatenate(
      [a_ref[0, 0] + a_ref[1, 0], a_ref[0, 1] + a_ref[1, 1]], axis=1)


def _tc_prep(degp, x, NP):
  def body(degp_ref, x_ref, dinv_ref, xp_ref):
    deg = degp_ref[0] + degp_ref[1] + 1.0
    dinv = 1.0 / jnp.sqrt(deg)
    dinv_ref[...] = dinv
    xp_ref[...] = x_ref[...] * dinv

  return pl.pallas_call(
      body,
      grid=(NP // BN,),
      in_specs=[_pair_spec((1,)), _node_spec((1,))],
      out_specs=[_node_spec((1,)), _node_spec((1,))],
      out_shape=[
          jax.ShapeDtypeStruct((NP, 1), F32),
          jax.ShapeDtypeStruct((NP, 1), F32),
      ],
  )(degp, x)


def _tc_mid1(acc1p, xp, dinv, W1, b1, W2, NP):
  def body(a_ref, xp_ref, dinv_ref, W1_ref, b1_ref, W2_ref, out_ref):
    dinv = dinv_ref[...]
    u = dinv * (a_ref[0] + a_ref[1] + xp_ref[...])
    h1 = jax.nn.relu(u * W1_ref[...] + b1_ref[...])
    hp2 = jnp.dot(h1, W2_ref[...], preferred_element_type=F32) * dinv
    out_ref[0] = hp2[:, :16]
    out_ref[1] = hp2[:, 16:]

  return pl.pallas_call(
      body,
      grid=(NP // BN,),
      in_specs=[
          _pair_spec((1,)), _node_spec((1,)), _node_spec((1,)),
          _full_spec((1, 32)), _full_spec((1, 32)), _full_spec((32, 32)),
      ],
      out_specs=_pair_spec((16,)),
      out_shape=jax.ShapeDtypeStruct((2, NP, 16), F32),
  )(acc1p, xp, dinv, W1, b1, W2)


def _tc_mid2(accp, hph, dinv, b, W_next, NP):
  def body(a_ref, hp_ref, dinv_ref, b_ref, W_ref, out_ref):
    dinv = dinv_ref[...]
    hp = jnp.concatenate([hp_ref[0], hp_ref[1]], axis=1)
    xn = jax.nn.relu(dinv * (_acc_sum(a_ref) + hp) + b_ref[...])
    hpn = jnp.dot(xn, W_ref[...], preferred_element_type=F32) * dinv
    out_ref[0] = hpn[:, :16]
    out_ref[1] = hpn[:, 16:]

  return pl.pallas_call(
      body,
      grid=(NP // BN,),
      in_specs=[
          _acc_spec(), _pair_spec((16,)), _node_spec((1,)),
          _full_spec((1, 32)), _full_spec((32, 32)),
      ],
      out_specs=_pair_spec((16,)),
      out_shape=jax.ShapeDtypeStruct((2, NP, 16), F32),
  )(accp, hph, dinv, b, W_next)


def _tc_final(accp, hph, dinv, b3, Wt1, bt1, Wt2, bt2, batch, Wg1, bg1, Wg2,
              bg2, NP, G):
  ngrid = NP // BN

  def body(a_ref, hp_ref, dinv_ref, b3_ref, Wt1_ref, bt1_ref, Wt2_ref,
           bt2_ref, batch_ref, Wg1_ref, bg1_ref, Wg2_ref, bg2_ref,
           theta_ref, bg_ref, sums_scr, cnts_scr):
    i = pl.program_id(0)
    dinv = dinv_ref[...]
    hp = jnp.concatenate([hp_ref[0], hp_ref[1]], axis=1)
    h3 = jax.nn.relu(dinv * (_acc_sum(a_ref) + hp) + b3_ref[...])

    t1 = jax.nn.relu(
        jnp.dot(h3, Wt1_ref[...], preferred_element_type=F32) + bt1_ref[...])
    t2 = jnp.dot(t1, Wt2_ref[...], preferred_element_type=F32) + bt2_ref[...]
    theta_ref[...] = jnp.pi * jax.nn.sigmoid(t2)

    @pl.when(i == 0)
    def _():
      sums_scr[...] = jnp.zeros((G, 32), F32)
      cnts_scr[...] = jnp.zeros((G, 1), F32)

    gids = jax.lax.broadcasted_iota(I32, (1, G), 1)
    mask = (batch_ref[...] == gids).astype(F32)
    dn = (((0,), (0,)), ((), ()))
    sums_scr[...] += jax.lax.dot_general(
        mask, h3, dn, preferred_element_type=F32)
    cnts_scr[...] += jax.lax.dot_general(
        mask, jnp.ones((BN, 1), F32), dn, preferred_element_type=F32)

    @pl.when(i == ngrid - 1)
    def _():
      emb = sums_scr[...] / jnp.maximum(cnts_scr[...], 1.0)
      g1 = jax.nn.relu(
          jnp.dot(emb, Wg1_ref[...], preferred_element_type=F32)
          + bg1_ref[...])
      g2 = jnp.dot(g1, Wg2_ref[...], preferred_element_type=F32) + bg2_ref[...]
      bg_ref[...] = 2.0 * jnp.pi * jax.nn.sigmoid(g2)

  return pl.pallas_call(
      body,
      grid=(ngrid,),
      in_specs=[
          _acc_spec(), _pair_spec((16,)), _node_spec((1,)),
          _full_spec((1, 32)), _full_spec((32, 32)), _full_spec((1, 32)),
          _full_spec((32, 1)), _full_spec((1, 1)), _node_spec((1,)),
          _full_spec((32, 32)), _full_spec((1, 32)), _full_spec((32, 2)),
          _full_spec((1, 2)),
      ],
      out_specs=[_node_spec((1,)), _full_spec((G, 2))],
      out_shape=[
          jax.ShapeDtypeStruct((NP, 1), F32),
          jax.ShapeDtypeStruct((G, 2), F32),
      ],
      scratch_shapes=[pltpu.VMEM((G, 32), F32), pltpu.VMEM((G, 1), F32)],
  )(accp, hph, dinv, b3, Wt1, bt1, Wt2, bt2, batch, Wg1, bg1, Wg2, bg2)


def kernel(x, edge_index, batch, W1, b1, W2, b2, W3, b3, Wt1, bt1, Wt2, bt2,
           Wg1, bg1, Wg2, bg2):
  N = x.shape[0]
  E = edge_index.shape[1]
  G = 64

  NP = -(-N // BN) * BN
  EP = -(-E // (NW * CH * 2)) * (NW * CH * 2)

  src = jnp.concatenate([edge_index[0], jnp.zeros((EP - E,), I32)])
  dst = jnp.concatenate(
      [edge_index[1], jnp.full((EP - E,), N, I32)])
  x_p = jnp.pad(x, ((0, NP - N), (0, 0)))
  batch_p = jnp.pad(batch, (0, NP - N), constant_values=G)[:, None]

  b1r, b2r, b3r = b1[None, :], b2[None, :], b3[None, :]
  bt1r, bt2r = bt1[None, :], bt2[None, :]
  bg1r, bg2r = bg1[None, :], bg2[None, :]

  degp = _make_sc_deg(NP, EP)(dst)
  dinv, xp = _tc_prep(degp[:, :, None], x_p, NP)

  acc1p = _make_sc_edge1(NP, EP)(xp[:, 0], src, dst)
  hph2 = _tc_mid1(acc1p[:, :, None], xp, dinv, W1, b1r, W2, NP)

  acc2p = _make_sc_edge(NP, EP, 16)(hph2[0], hph2[1], src, dst)
  hph3 = _tc_mid2(acc2p, hph2, dinv, b2r, W3, NP)

  acc3p = _make_sc_edge(NP, EP, 16)(hph3[0], hph3[1], src, dst)
  theta_p, beta_gamma = _tc_final(acc3p, hph3, dinv, b3r, Wt1, bt1r, Wt2,
                                  bt2r, batch_p, Wg1, bg1r, Wg2, bg2r, NP, G)

  return (theta_p[:N, 0], beta_gamma)


# speedup vs baseline: 1.2801x; 1.2801x over previous
import functools

import jax
import jax.numpy as jnp
from jax import lax
from jax.experimental import pallas as pl
from jax.experimental.pallas import tpu as pltpu
from jax.experimental.pallas import tpu_sc as plsc

F32 = jnp.float32
I32 = jnp.int32
NSC = 2
NSUB = 16
NW = NSC * NSUB
CH = 1024
ZCH = 448


def _mesh():
  return plsc.VectorSubcoreMesh(core_axis_name="c", subcore_axis_name="s")


def _ids():
  c = lax.axis_index("c")
  s = lax.axis_index("s")
  return c, s, c * NSUB + s


@functools.cache
def _make_sc_deg(NP, EP):
  TPE = EP // NW
  NSTR = TPE // CH
  STRIPE = NP // NSUB

  def body(dst_hbm, out_hbm, idx_v, ones_v, zbuf_v, acc_sh):
    c, s, w = _ids()

    @pl.loop(0, CH // 16)
    def _(i):
      ones_v[pl.ds(i * 16, 16)] = jnp.ones((16,), F32)

    @pl.loop(0, ZCH // 16)
    def _(i):
      zbuf_v[pl.ds(i * 16, 16)] = jnp.zeros((16,), F32)

    @pl.loop(0, STRIPE // ZCH)
    def _(i):
      pltpu.sync_copy(zbuf_v, acc_sh.at[pl.ds(s * STRIPE + i * ZCH, ZCH)])

    plsc.subcore_barrier()

    @pl.loop(0, NSTR)
    def _(j):
      pltpu.sync_copy(dst_hbm.at[pl.ds(w * TPE + j * CH, CH)], idx_v)
      pltpu.sync_copy(ones_v, acc_sh.at[idx_v], add=True)

    plsc.subcore_barrier()
    pltpu.sync_copy(
        acc_sh.at[pl.ds(s * STRIPE, STRIPE)],
        out_hbm.at[c, pl.ds(s * STRIPE, STRIPE)],
    )

  return pl.kernel(
      body,
      out_type=jax.ShapeDtypeStruct((NSC, NP), F32),
      mesh=_mesh(),
      compiler_params=pltpu.CompilerParams(use_tc_tiling_on_sc=False),
      scratch_types=[
          pltpu.VMEM((CH,), I32),
          pltpu.VMEM((CH,), F32),
          pltpu.VMEM((ZCH,), F32),
          pltpu.VMEM_SHARED((NP,), F32),
      ],
  )


@functools.cache
def _make_sc_edge1(NP, EP):
  TPE = EP // NW
  NSTR = TPE // CH
  STRIPE = NP // NSUB

  def body(xp_hbm, src_hbm, dst_hbm, out_hbm, sidx_v, didx_v, val_v, zbuf_v,
           xp_sh, acc_sh):
    c, s, w = _ids()

    @pl.loop(0, ZCH // 16)
    def _(i):
      zbuf_v[pl.ds(i * 16, 16)] = jnp.zeros((16,), F32)

    pltpu.sync_copy(
        xp_hbm.at[pl.ds(s * STRIPE, STRIPE)],
        xp_sh.at[pl.ds(s * STRIPE, STRIPE)],
    )

    @pl.loop(0, STRIPE // ZCH)
    def _(i):
      pltpu.sync_copy(zbuf_v, acc_sh.at[pl.ds(s * STRIPE + i * ZCH, ZCH)])

    plsc.subcore_barrier()

    @pl.loop(0, NSTR)
    def _(j):
      base = w * TPE + j * CH
      pltpu.sync_copy(src_hbm.at[pl.ds(base, CH)], sidx_v)
      pltpu.sync_copy(xp_sh.at[sidx_v], val_v)
      pltpu.sync_copy(dst_hbm.at[pl.ds(base, CH)], didx_v)
      pltpu.sync_copy(val_v, acc_sh.at[didx_v], add=True)

    plsc.subcore_barrier()
    pltpu.sync_copy(
        acc_sh.at[pl.ds(s * STRIPE, STRIPE)],
        out_hbm.at[c, pl.ds(s * STRIPE, STRIPE)],
    )

  return pl.kernel(
      body,
      out_type=jax.ShapeDtypeStruct((NSC, NP), F32),
      mesh=_mesh(),
      compiler_params=pltpu.CompilerParams(use_tc_tiling_on_sc=False),
      scratch_types=[
          pltpu.VMEM((CH,), I32),
          pltpu.VMEM((CH,), I32),
          pltpu.VMEM((CH,), F32),
          pltpu.VMEM((ZCH,), F32),
          pltpu.VMEM_SHARED((NP,), F32),
          pltpu.VMEM_SHARED((NP,), F32),
      ],
  )


@functools.cache
def _make_sc_edge(NP, EP, HH):
  CHE = 1024
  TPE = EP // NW
  NSTR = TPE // CHE
  STRIPE = NP // NSUB
  ZR = 448

  def body(hp0_hbm, hp1_hbm, src_hbm, dst_hbm, out_hbm, sidx_v, didx_v,
           rows_v, zbuf_v, acc_sh, sem):
    c, s, w = _ids()

    @pl.loop(0, ZR)
    def _(r):
      zbuf_v[r] = jnp.zeros((HH,), F32)

    for f in range(2):
      hp_hbm = hp0_hbm if f == 0 else hp1_hbm

      @pl.loop(0, STRIPE // ZR)
      def _(i):
        pltpu.sync_copy(zbuf_v, acc_sh.at[pl.ds(s * STRIPE + i * ZR, ZR)])

      plsc.subcore_barrier()

      @pl.loop(0, NSTR)
      def _(j):
        base = w * TPE + j * CHE
        pltpu.sync_copy(src_hbm.at[pl.ds(base, CHE)], sidx_v)
        pltpu.async_copy(hp_hbm.at[sidx_v], rows_v, sem).wait()
        pltpu.sync_copy(dst_hbm.at[pl.ds(base, CHE)], didx_v)
        pltpu.sync_copy(rows_v, acc_sh.at[didx_v], add=True)

      plsc.subcore_barrier()
      pltpu.sync_copy(
          acc_sh.at[pl.ds(s * STRIPE, STRIPE)],
          out_hbm.at[c, f, pl.ds(s * STRIPE, STRIPE)],
      )
      plsc.subcore_barrier()

  return pl.kernel(
      body,
      out_type=jax.ShapeDtypeStruct((NSC, 2, NP, HH), F32),
      mesh=_mesh(),
      compiler_params=pltpu.CompilerParams(use_tc_tiling_on_sc=False),
      scratch_types=[
          pltpu.VMEM((CHE,), I32),
          pltpu.VMEM((CHE,), I32),
          pltpu.VMEM((CHE, HH), F32),
          pltpu.VMEM((ZR, HH), F32),
          pltpu.VMEM_SHARED((NP, HH), F32),
          pltpu.SemaphoreType.DMA,
      ],
  )


BN = 2048


def _node_spec(shape_tail):
  return pl.BlockSpec((BN,) + shape_tail, lambda i: (i,) + (0,) * len(shape_tail))


def _pair_spec(shape_tail):
  return pl.BlockSpec((NSC, BN) + shape_tail,
                      lambda i: (0, i) + (0,) * len(shape_tail))


def _full_spec(shape):
  return pl.BlockSpec(shape, lambda i: (0,) * len(shape))


def _acc_spec():
  return pl.BlockSpec((NSC, 2, BN, 16), lambda i: (0, 0, i, 0))


def _acc_sum(a_ref):
  return jnp.concatenate(
      [a_ref[0, 0] + a_ref[1, 0], a_ref[0, 1] + a_ref[1, 1]], axis=1)


def _tc_prep(degp, x, NP):
  def body(degp_ref, x_ref, dinv_ref, xp_ref):
    deg = degp_ref[0] + degp_ref[1] + 1.0
    dinv = 1.0 / jnp.sqrt(deg)
    dinv_ref[...] = dinv
    xp_ref[...] = x_ref[...] * dinv

  return pl.pallas_call(
      body,
      grid=(NP // BN,),
      in_specs=[_pair_spec((1,)), _node_spec((1,))],
      out_specs=[_node_spec((1,)), _node_spec((1,))],
      out_shape=[
          jax.ShapeDtypeStruct((NP, 1), F32),
          jax.ShapeDtypeStruct((NP, 1), F32),
      ],
  )(degp, x)


def _tc_mid1(acc1p, xp, dinv, W1, b1, W2, NP):
  def body(a_ref, xp_ref, dinv_ref, W1_ref, b1_ref, W2_ref, out_ref):
    dinv = dinv_ref[...]
    u = dinv * (a_ref[0] + a_ref[1] + xp_ref[...])
    h1 = jax.nn.relu(u * W1_ref[...] + b1_ref[...])
    hp2 = jnp.dot(h1, W2_ref[...], preferred_element_type=F32) * dinv
    out_ref[0] = hp2[:, :16]
    out_ref[1] = hp2[:, 16:]

  return pl.pallas_call(
      body,
      grid=(NP // BN,),
      in_specs=[
          _pair_spec((1,)), _node_spec((1,)), _node_spec((1,)),
          _full_spec((1, 32)), _full_spec((1, 32)), _full_spec((32, 32)),
      ],
      out_specs=_pair_spec((16,)),
      out_shape=jax.ShapeDtypeStruct((2, NP, 16), F32),
  )(acc1p, xp, dinv, W1, b1, W2)


def _tc_mid2(accp, hph, dinv, b, W_next, NP):
  def body(a_ref, hp_ref, dinv_ref, b_ref, W_ref, out_ref):
    dinv = dinv_ref[...]
    hp = jnp.concatenate([hp_ref[0], hp_ref[1]], axis=1)
    xn = jax.nn.relu(dinv * (_acc_sum(a_ref) + hp) + b_ref[...])
    hpn = jnp.dot(xn, W_ref[...], preferred_element_type=F32) * dinv
    out_ref[0] = hpn[:, :16]
    out_ref[1] = hpn[:, 16:]

  return pl.pallas_call(
      body,
      grid=(NP // BN,),
      in_specs=[
          _acc_spec(), _pair_spec((16,)), _node_spec((1,)),
          _full_spec((1, 32)), _full_spec((32, 32)),
      ],
      out_specs=_pair_spec((16,)),
      out_shape=jax.ShapeDtypeStruct((2, NP, 16), F32),
  )(accp, hph, dinv, b, W_next)


def _tc_final(accp, hph, dinv, b3, Wt1, bt1, Wt2, bt2, batch, Wg1, bg1, Wg2,
              bg2, NP, G):
  ngrid = NP // BN

  def body(a_ref, hp_ref, dinv_ref, b3_ref, Wt1_ref, bt1_ref, Wt2_ref,
           bt2_ref, batch_ref, Wg1_ref, bg1_ref, Wg2_ref, bg2_ref,
           theta_ref, bg_ref, sums_scr, cnts_scr):
    i = pl.program_id(0)
    dinv = dinv_ref[...]
    hp = jnp.concatenate([hp_ref[0], hp_ref[1]], axis=1)
    h3 = jax.nn.relu(dinv * (_acc_sum(a_ref) + hp) + b3_ref[...])

    t1 = jax.nn.relu(
        jnp.dot(h3, Wt1_ref[...], preferred_element_type=F32) + bt1_ref[...])
    t2 = jnp.dot(t1, Wt2_ref[...], preferred_element_type=F32) + bt2_ref[...]
    theta_ref[...] = jnp.pi * jax.nn.sigmoid(t2)

    @pl.when(i == 0)
    def _():
      sums_scr[...] = jnp.zeros((G, 32), F32)
      cnts_scr[...] = jnp.zeros((G, 1), F32)

    gids = jax.lax.broadcasted_iota(I32, (1, G), 1)
    mask = (batch_ref[...] == gids).astype(F32)
    dn = (((0,), (0,)), ((), ()))
    sums_scr[...] += jax.lax.dot_general(
        mask, h3, dn, preferred_element_type=F32)
    cnts_scr[...] += jax.lax.dot_general(
        mask, jnp.ones((BN, 1), F32), dn, preferred_element_type=F32)

    @pl.when(i == ngrid - 1)
    def _():
      emb = sums_scr[...] / jnp.maximum(cnts_scr[...], 1.0)
      g1 = jax.nn.relu(
          jnp.dot(emb, Wg1_ref[...], preferred_element_type=F32)
          + bg1_ref[...])
      g2 = jnp.dot(g1, Wg2_ref[...], preferred_element_type=F32) + bg2_ref[...]
      bg_ref[...] = 2.0 * jnp.pi * jax.nn.sigmoid(g2)

  return pl.pallas_call(
      body,
      grid=(ngrid,),
      in_specs=[
          _acc_spec(), _pair_spec((16,)), _node_spec((1,)),
          _full_spec((1, 32)), _full_spec((32, 32)), _full_spec((1, 32)),
          _full_spec((32, 1)), _full_spec((1, 1)), _node_spec((1,)),
          _full_spec((32, 32)), _full_spec((1, 32)), _full_spec((32, 2)),
          _full_spec((1, 2)),
      ],
      out_specs=[_node_spec((1,)), _full_spec((G, 2))],
      out_shape=[
          jax.ShapeDtypeStruct((NP, 1), F32),
          jax.ShapeDtypeStruct((G, 2), F32),
      ],
      scratch_shapes=[pltpu.VMEM((G, 32), F32), pltpu.VMEM((G, 1), F32)],
  )(accp, hph, dinv, b3, Wt1, bt1, Wt2, bt2, batch, Wg1, bg1, Wg2, bg2)


def kernel(x, edge_index, batch, W1, b1, W2, b2, W3, b3, Wt1, bt1, Wt2, bt2,
           Wg1, bg1, Wg2, bg2):
  N = x.shape[0]
  E = edge_index.shape[1]
  G = 64

  NP = -(-N // BN) * BN
  EP = -(-E // (NW * CH)) * (NW * CH)

  src = jnp.concatenate([edge_index[0], jnp.zeros((EP - E,), I32)])
  dst = jnp.concatenate(
      [edge_index[1], jnp.full((EP - E,), N, I32)])
  x_p = jnp.pad(x, ((0, NP - N), (0, 0)))
  batch_p = jnp.pad(batch, (0, NP - N), constant_values=G)[:, None]

  b1r, b2r, b3r = b1[None, :], b2[None, :], b3[None, :]
  bt1r, bt2r = bt1[None, :], bt2[None, :]
  bg1r, bg2r = bg1[None, :], bg2[None, :]

  degp = _make_sc_deg(NP, EP)(dst)
  dinv, xp = _tc_prep(degp[:, :, None], x_p, NP)

  acc1p = _make_sc_edge1(NP, EP)(xp[:, 0], src, dst)
  hph2 = _tc_mid1(acc1p[:, :, None], xp, dinv, W1, b1r, W2, NP)

  acc2p = _make_sc_edge(NP, EP, 16)(hph2[0], hph2[1], src, dst)
  hph3 = _tc_mid2(acc2p, hph2, dinv, b2r, W3, NP)

  acc3p = _make_sc_edge(NP, EP, 16)(hph3[0], hph3[1], src, dst)
  theta_p, beta_gamma = _tc_final(acc3p, hph3, dinv, b3r, Wt1, bt1r, Wt2,
                                  bt2r, batch_p, Wg1, bg1r, Wg2, bg2r, NP, G)

  return (theta_p[:N, 0], beta_gamma)


# speedup vs baseline: 1.3186x; 1.0301x over previous
import functools

import jax
import jax.numpy as jnp
from jax import lax
from jax.experimental import pallas as pl
from jax.experimental.pallas import tpu as pltpu
from jax.experimental.pallas import tpu_sc as plsc

F32 = jnp.float32
I32 = jnp.int32
NSC = 2
NSUB = 16
NW = NSC * NSUB
CH = 1024
ZCH = 448


def _mesh():
  return plsc.VectorSubcoreMesh(core_axis_name="c", subcore_axis_name="s")


def _ids():
  c = lax.axis_index("c")
  s = lax.axis_index("s")
  return c, s, c * NSUB + s


@functools.cache
def _make_sc_head(NP, EP):
  TPE = EP // NW
  TPEF = EP // NSUB
  NSTR = TPE // CH
  NSTRF = TPEF // CH
  STRIPE = NP // NSUB

  def body(x_hbm, src_hbm, dst_hbm, dinv_hbm, xp_hbm, acc_hbm, sidx_v,
           didx_v, val_v, zbuf_v, deg_stripe, x_stripe, deg_sh, xp_sh,
           acc_sh):
    c, s, w = _ids()

    @pl.loop(0, ZCH // 16)
    def _(i):
      zbuf_v[pl.ds(i * 16, 16)] = jnp.zeros((16,), F32)

    @pl.loop(0, STRIPE // ZCH)
    def _(i):
      pltpu.sync_copy(zbuf_v, deg_sh.at[pl.ds(s * STRIPE + i * ZCH, ZCH)])
      pltpu.sync_copy(zbuf_v, acc_sh.at[pl.ds(s * STRIPE + i * ZCH, ZCH)])

    @pl.loop(0, CH // 16)
    def _(i):
      val_v[pl.ds(i * 16, 16)] = jnp.ones((16,), F32)

    plsc.subcore_barrier()

    @pl.loop(0, NSTRF)
    def _(j):
      pltpu.sync_copy(dst_hbm.at[pl.ds(s * TPEF + j * CH, CH)], didx_v)
      pltpu.sync_copy(val_v, deg_sh.at[didx_v], add=True)

    plsc.subcore_barrier()

    r0 = s * STRIPE
    pltpu.sync_copy(deg_sh.at[pl.ds(r0, STRIPE)], deg_stripe)
    pltpu.sync_copy(x_hbm.at[pl.ds(r0, STRIPE)], x_stripe)

    @pl.loop(0, STRIPE // 16)
    def _(r):
      d = deg_stripe[pl.ds(r * 16, 16)] + 1.0
      u = 0.25 * d + 1.0
      for _ in range(12):
        u = 0.5 * (u + d / u)
      y = 1.0 / u
      deg_stripe[pl.ds(r * 16, 16)] = y
      x_stripe[pl.ds(r * 16, 16)] = x_stripe[pl.ds(r * 16, 16)] * y

    pltpu.sync_copy(x_stripe, xp_sh.at[pl.ds(r0, STRIPE)])

    @pl.when(c == 0)
    def _():
      pltpu.sync_copy(deg_stripe, dinv_hbm.at[pl.ds(r0, STRIPE)])
      pltpu.sync_copy(x_stripe, xp_hbm.at[pl.ds(r0, STRIPE)])

    plsc.subcore_barrier()

    @pl.loop(0, NSTR)
    def _(j):
      base = w * TPE + j * CH
      pltpu.sync_copy(src_hbm.at[pl.ds(base, CH)], sidx_v)
      pltpu.sync_copy(xp_sh.at[sidx_v], val_v)
      pltpu.sync_copy(dst_hbm.at[pl.ds(base, CH)], didx_v)
      pltpu.sync_copy(val_v, acc_sh.at[didx_v], add=True)

    plsc.subcore_barrier()
    pltpu.sync_copy(
        acc_sh.at[pl.ds(s * STRIPE, STRIPE)],
        acc_hbm.at[c, pl.ds(s * STRIPE, STRIPE)],
    )

  return pl.kernel(
      body,
      out_type=(
          jax.ShapeDtypeStruct((NP,), F32),
          jax.ShapeDtypeStruct((NP,), F32),
          jax.ShapeDtypeStruct((NSC, NP), F32),
      ),
      mesh=_mesh(),
      compiler_params=pltpu.CompilerParams(use_tc_tiling_on_sc=False),
      scratch_types=[
          pltpu.VMEM((CH,), I32),
          pltpu.VMEM((CH,), I32),
          pltpu.VMEM((CH,), F32),
          pltpu.VMEM((ZCH,), F32),
          pltpu.VMEM((NP // NSUB,), F32),
          pltpu.VMEM((NP // NSUB,), F32),
          pltpu.VMEM_SHARED((NP,), F32),
          pltpu.VMEM_SHARED((NP,), F32),
          pltpu.VMEM_SHARED((NP,), F32),
      ],
  )


@functools.cache
def _make_sc_edge(NP, EP, HH):
  CHE = 1024
  TPE = EP // NW
  NSTR = TPE // CHE
  STRIPE = NP // NSUB
  ZR = 448

  def body(hp0_hbm, hp1_hbm, src_hbm, dst_hbm, out_hbm, sidx_v, didx_v,
           rows_v, zbuf_v, acc_sh, sem):
    c, s, w = _ids()

    @pl.loop(0, ZR)
    def _(r):
      zbuf_v[r] = jnp.zeros((HH,), F32)

    for f in range(2):
      hp_hbm = hp0_hbm if f == 0 else hp1_hbm

      @pl.loop(0, STRIPE // ZR)
      def _(i):
        pltpu.sync_copy(zbuf_v, acc_sh.at[pl.ds(s * STRIPE + i * ZR, ZR)])

      plsc.subcore_barrier()

      @pl.loop(0, NSTR)
      def _(j):
        base = w * TPE + j * CHE
        pltpu.sync_copy(src_hbm.at[pl.ds(base, CHE)], sidx_v)
        pltpu.async_copy(hp_hbm.at[sidx_v], rows_v, sem).wait()
        pltpu.sync_copy(dst_hbm.at[pl.ds(base, CHE)], didx_v)
        pltpu.sync_copy(rows_v, acc_sh.at[didx_v], add=True)

      plsc.subcore_barrier()
      pltpu.sync_copy(
          acc_sh.at[pl.ds(s * STRIPE, STRIPE)],
          out_hbm.at[c, f, pl.ds(s * STRIPE, STRIPE)],
      )
      plsc.subcore_barrier()

  return pl.kernel(
      body,
      out_type=jax.ShapeDtypeStruct((NSC, 2, NP, HH), F32),
      mesh=_mesh(),
      compiler_params=pltpu.CompilerParams(use_tc_tiling_on_sc=False),
      scratch_types=[
          pltpu.VMEM((CHE,), I32),
          pltpu.VMEM((CHE,), I32),
          pltpu.VMEM((CHE, HH), F32),
          pltpu.VMEM((ZR, HH), F32),
          pltpu.VMEM_SHARED((NP, HH), F32),
          pltpu.SemaphoreType.DMA,
      ],
  )


BN = 2048


def _node_spec(shape_tail):
  return pl.BlockSpec((BN,) + shape_tail, lambda i: (i,) + (0,) * len(shape_tail))


def _pair_spec(shape_tail):
  return pl.BlockSpec((NSC, BN) + shape_tail,
                      lambda i: (0, i) + (0,) * len(shape_tail))


def _full_spec(shape):
  return pl.BlockSpec(shape, lambda i: (0,) * len(shape))


def _acc_spec():
  return pl.BlockSpec((NSC, 2, BN, 16), lambda i: (0, 0, i, 0))


def _acc_sum(a_ref):
  return jnp.concatenate(
      [a_ref[0, 0] + a_ref[1, 0], a_ref[0, 1] + a_ref[1, 1]], axis=1)


def _tc_mid1(acc1p, xp, dinv, W1, b1, W2, NP):
  def body(a_ref, xp_ref, dinv_ref, W1_ref, b1_ref, W2_ref, out_ref):
    dinv = dinv_ref[...]
    u = dinv * (a_ref[0] + a_ref[1] + xp_ref[...])
    h1 = jax.nn.relu(u * W1_ref[...] + b1_ref[...])
    hp2 = jnp.dot(h1, W2_ref[...], preferred_element_type=F32) * dinv
    out_ref[0] = hp2[:, :16]
    out_ref[1] = hp2[:, 16:]

  return pl.pallas_call(
      body,
      grid=(NP // BN,),
      in_specs=[
          _pair_spec((1,)), _node_spec((1,)), _node_spec((1,)),
          _full_spec((1, 32)), _full_spec((1, 32)), _full_spec((32, 32)),
      ],
      out_specs=_pair_spec((16,)),
      out_shape=jax.ShapeDtypeStruct((2, NP, 16), F32),
  )(acc1p, xp, dinv, W1, b1, W2)


def _tc_mid2(accp, hph, dinv, b, W_next, NP):
  def body(a_ref, hp_ref, dinv_ref, b_ref, W_ref, out_ref):
    dinv = dinv_ref[...]
    hp = jnp.concatenate([hp_ref[0], hp_ref[1]], axis=1)
    xn = jax.nn.relu(dinv * (_acc_sum(a_ref) + hp) + b_ref[...])
    hpn = jnp.dot(xn, W_ref[...], preferred_element_type=F32) * dinv
    out_ref[0] = hpn[:, :16]
    out_ref[1] = hpn[:, 16:]

  return pl.pallas_call(
      body,
      grid=(NP // BN,),
      in_specs=[
          _acc_spec(), _pair_spec((16,)), _node_spec((1,)),
          _full_spec((1, 32)), _full_spec((32, 32)),
      ],
      out_specs=_pair_spec((16,)),
      out_shape=jax.ShapeDtypeStruct((2, NP, 16), F32),
  )(accp, hph, dinv, b, W_next)


def _tc_final(accp, hph, dinv, b3, Wt1, bt1, Wt2, bt2, batch, Wg1, bg1, Wg2,
              bg2, NP, G):
  ngrid = NP // BN

  def body(a_ref, hp_ref, dinv_ref, b3_ref, Wt1_ref, bt1_ref, Wt2_ref,
           bt2_ref, batch_ref, Wg1_ref, bg1_ref, Wg2_ref, bg2_ref,
           theta_ref, bg_ref, sums_scr, cnts_scr):
    i = pl.program_id(0)
    dinv = dinv_ref[...]
    hp = jnp.concatenate([hp_ref[0], hp_ref[1]], axis=1)
    h3 = jax.nn.relu(dinv * (_acc_sum(a_ref) + hp) + b3_ref[...])

    t1 = jax.nn.relu(
        jnp.dot(h3, Wt1_ref[...], preferred_element_type=F32) + bt1_ref[...])
    t2 = jnp.dot(t1, Wt2_ref[...], preferred_element_type=F32) + bt2_ref[...]
    theta_ref[...] = jnp.pi * jax.nn.sigmoid(t2)

    @pl.when(i == 0)
    def _():
      sums_scr[...] = jnp.zeros((G, 32), F32)
      cnts_scr[...] = jnp.zeros((G, 1), F32)

    gids = jax.lax.broadcasted_iota(I32, (1, G), 1)
    mask = (batch_ref[...] == gids).astype(F32)
    dn = (((0,), (0,)), ((), ()))
    sums_scr[...] += jax.lax.dot_general(
        mask, h3, dn, preferred_element_type=F32)
    cnts_scr[...] += jax.lax.dot_general(
        mask, jnp.ones((BN, 1), F32), dn, preferred_element_type=F32)

    @pl.when(i == ngrid - 1)
    def _():
      emb = sums_scr[...] / jnp.maximum(cnts_scr[...], 1.0)
      g1 = jax.nn.relu(
          jnp.dot(emb, Wg1_ref[...], preferred_element_type=F32)
          + bg1_ref[...])
      g2 = jnp.dot(g1, Wg2_ref[...], preferred_element_type=F32) + bg2_ref[...]
      bg_ref[...] = 2.0 * jnp.pi * jax.nn.sigmoid(g2)

  return pl.pallas_call(
      body,
      grid=(ngrid,),
      in_specs=[
          _acc_spec(), _pair_spec((16,)), _node_spec((1,)),
          _full_spec((1, 32)), _full_spec((32, 32)), _full_spec((1, 32)),
          _full_spec((32, 1)), _full_spec((1, 1)), _node_spec((1,)),
          _full_spec((32, 32)), _full_spec((1, 32)), _full_spec((32, 2)),
          _full_spec((1, 2)),
      ],
      out_specs=[_node_spec((1,)), _full_spec((G, 2))],
      out_shape=[
          jax.ShapeDtypeStruct((NP, 1), F32),
          jax.ShapeDtypeStruct((G, 2), F32),
      ],
      scratch_shapes=[pltpu.VMEM((G, 32), F32), pltpu.VMEM((G, 1), F32)],
  )(accp, hph, dinv, b3, Wt1, bt1, Wt2, bt2, batch, Wg1, bg1, Wg2, bg2)


def kernel(x, edge_index, batch, W1, b1, W2, b2, W3, b3, Wt1, bt1, Wt2, bt2,
           Wg1, bg1, Wg2, bg2):
  N = x.shape[0]
  E = edge_index.shape[1]
  G = 64

  NP = -(-N // BN) * BN
  EP = -(-E // (NW * CH)) * (NW * CH)

  src = jnp.concatenate([edge_index[0], jnp.zeros((EP - E,), I32)])
  dst = jnp.concatenate(
      [edge_index[1], jnp.full((EP - E,), N, I32)])
  x_p = jnp.pad(x, ((0, NP - N), (0, 0)))
  batch_p = jnp.pad(batch, (0, NP - N), constant_values=G)[:, None]

  b1r, b2r, b3r = b1[None, :], b2[None, :], b3[None, :]
  bt1r, bt2r = bt1[None, :], bt2[None, :]
  bg1r, bg2r = bg1[None, :], bg2[None, :]

  dinv, xp, acc1p = _make_sc_head(NP, EP)(x_p[:, 0], src, dst)
  dinv = dinv[:, None]
  hph2 = _tc_mid1(acc1p[:, :, None], xp[:, None], dinv, W1, b1r, W2, NP)

  acc2p = _make_sc_edge(NP, EP, 16)(hph2[0], hph2[1], src, dst)
  hph3 = _tc_mid2(acc2p, hph2, dinv, b2r, W3, NP)

  acc3p = _make_sc_edge(NP, EP, 16)(hph3[0], hph3[1], src, dst)
  theta_p, beta_gamma = _tc_final(acc3p, hph3, dinv, b3r, Wt1, bt1r, Wt2,
                                  bt2r, batch_p, Wg1, bg1r, Wg2, bg2r, NP, G)

  return (theta_p[:N, 0], beta_gamma)
